# baseline (speedup 1.0000x reference)
import functools

import jax
import jax.numpy as jnp
from jax import lax
from jax.experimental import pallas as pl
from jax.experimental.pallas import tpu as pltpu
from jax.experimental.pallas import tpu_sc as plsc

T = 2048
D = 768
F = 1024
E = 16
NW = 32
RPW = T // NW
BM = 128
NC = T // BM
CH = 512


def _silu(v):
    return v * (1.0 / (1.0 + jnp.exp(-v)))


def _gate_meta_body(x_ref, gw_ref, pos_ref, offs_ref, wrow_ref):
    x = x_ref[...]
    logits = jnp.dot(x, gw_ref[...].T, preferred_element_type=jnp.float32)
    m = jnp.max(logits, axis=1, keepdims=True)
    p = jnp.exp(logits - m)
    scores = p / jnp.sum(p, axis=1, keepdims=True)
    eidx = lax.broadcasted_iota(jnp.int32, (T, E), 1)
    first = jnp.min(jnp.where(logits == m, eidx, E), axis=1, keepdims=True)
    one_hot = (eidx == first).astype(jnp.float32)
    w = jnp.sum(scores * one_hot, axis=1, keepdims=True)
    wrow_ref[...] = jnp.broadcast_to(w, (T, 128))

    counts = jnp.sum(one_hot, axis=0, keepdims=True)
    iE = lax.broadcasted_iota(jnp.int32, (E, E), 0)
    jE = lax.broadcasted_iota(jnp.int32, (E, E), 1)
    Ue = (iE < jE).astype(jnp.float32)
    offs_row = jnp.dot(jnp.broadcast_to(counts, (8, E)), Ue,
                       preferred_element_type=jnp.float32)[0:1, :]
    Lc = (jE < iE).astype(jnp.float32)
    chunk_tot = jnp.sum(one_hot.reshape(NC, BM, E), axis=1)
    chunk_excl = jnp.dot(Lc, chunk_tot, preferred_element_type=jnp.float32)
    iB = lax.broadcasted_iota(jnp.int32, (BM, BM), 0)
    jB = lax.broadcasted_iota(jnp.int32, (BM, BM), 1)
    Lb = (jB < iB).astype(jnp.float32)
    for c in range(NC):
        oh_c = one_hot.reshape(NC, BM, E)[c]
        within = jnp.dot(Lb, oh_c, preferred_element_type=jnp.float32)
        rank = jnp.sum((within + chunk_excl[c:c + 1, :]) * oh_c, axis=1,
                       keepdims=True)
        base = jnp.sum(offs_row * oh_c, axis=1, keepdims=True)
        pos_c = (rank + base).astype(jnp.int32)
        rows_per_chunk = BM // RPW
        pos_ref[pl.ds(c * rows_per_chunk, rows_per_chunk), :] = (
            pos_c.reshape(rows_per_chunk, RPW))

    offs_pad = jnp.concatenate(
        [offs_row, jnp.full((1, E), float(T), jnp.float32)], axis=1)
    offs_ref[...] = offs_pad.astype(jnp.int32)


def _gate_meta(x, gate_w):
    return pl.pallas_call(
        _gate_meta_body,
        out_shape=(jax.ShapeDtypeStruct((NW, RPW), jnp.int32),
                   jax.ShapeDtypeStruct((1, 2 * E), jnp.int32),
                   jax.ShapeDtypeStruct((T, 128), jnp.float32)),
    )(x, gate_w)


@functools.cache
def _sc_kernels():
    mesh = plsc.VectorSubcoreMesh(core_axis_name="c", subcore_axis_name="s")

    @functools.partial(
        pl.kernel,
        out_type=(jax.ShapeDtypeStruct((T, D), jnp.float32),
                  jax.ShapeDtypeStruct((T, 128), jnp.float32)),
        mesh=mesh,
        scratch_types=[
            pltpu.VMEM((RPW,), jnp.int32),
            pltpu.VMEM((RPW, D), jnp.float32),
            pltpu.VMEM((RPW, 128), jnp.float32),
            pltpu.SemaphoreType.DMA,
            pltpu.SemaphoreType.DMA,
        ],
    )
    def _sc_scatter(x_hbm, w_hbm, pos_hbm, sx_hbm, sw_hbm,
                    idx_v, x_v, w_v, s1, s2):
        wid = lax.axis_index("s") * 2 + lax.axis_index("c")
        base = wid * RPW
        pltpu.sync_copy(pos_hbm.at[wid], idx_v)
        pltpu.sync_copy(x_hbm.at[pl.ds(base, RPW)], x_v)
        pltpu.sync_copy(w_hbm.at[pl.ds(base, RPW)], w_v)
        c1 = pltpu.async_copy(x_v, sx_hbm.at[idx_v], s1)
        c2 = pltpu.async_copy(w_v, sw_hbm.at[idx_v], s2)
        c1.wait()
        c2.wait()

    @functools.partial(
        pl.kernel,
        out_type=jax.ShapeDtypeStruct((T, D), jnp.float32),
        mesh=mesh,
        scratch_types=[
            pltpu.VMEM((RPW,), jnp.int32),
            pltpu.VMEM((RPW, D), jnp.float32),
            pltpu.SemaphoreType.DMA,
        ],
    )
    def _sc_gather(sy_hbm, pos_hbm, out_hbm, idx_v, y_v, sem):
        wid = lax.axis_index("s") * 2 + lax.axis_index("c")
        base = wid * RPW
        pltpu.sync_copy(pos_hbm.at[wid], idx_v)
        pltpu.async_copy(sy_hbm.at[idx_v], y_v, sem).wait()
        pltpu.sync_copy(y_v, out_hbm.at[pl.ds(base, RPW)])

    return _sc_scatter, _sc_gather


def _ffn_body(offs_ref, sx_ref, sw_ref, w1_ref, b1_ref, w2_ref,
              b2_ref, w3_ref, b3_ref, sw1_ref, sb1_ref, sw2_ref, sb2_ref,
              sw3_ref, sb3_ref, out_ref):
    e = pl.program_id(0)

    @pl.when(e == 0)
    def _init():
        out_ref[...] = jnp.zeros_like(out_ref)

    for c in range(NC):
        @pl.when(e == c)
        def _shared(c0=c * BM):
            xs = sx_ref[pl.ds(c0, BM), :]
            g1 = jnp.dot(xs, sw1_ref[...].T, preferred_element_type=jnp.float32) + sb1_ref[...]
            g3 = jnp.dot(xs, sw3_ref[...].T, preferred_element_type=jnp.float32) + sb3_ref[...]
            z = jnp.dot(_silu(g1) * g3, sw2_ref[...].T,
                        preferred_element_type=jnp.float32) + sb2_ref[...]
            out_ref[pl.ds(c0, BM), :] += z

    start = offs_ref[e]
    end = offs_ref[e + 1]
    for c in range(NC):
        c0 = c * BM

        @pl.when((end > c0) & (start < c0 + BM))
        def _chunk(c0=c0):
            xs = sx_ref[pl.ds(c0, BM), :]
            h1 = jnp.dot(xs, w1_ref[0].T, preferred_element_type=jnp.float32) + b1_ref[0]
            h3 = jnp.dot(xs, w3_ref[0].T, preferred_element_type=jnp.float32) + b3_ref[0]
            h = _silu(h1) * h3
            eo = jnp.dot(h, w2_ref[0].T, preferred_element_type=jnp.float32) + b2_ref[0]
            rows = c0 + lax.broadcasted_iota(jnp.int32, (BM, 1), 0)
            msk = (rows >= start) & (rows < end)
            wcol = sw_ref[pl.ds(c0, BM), 0:1]
            out_ref[pl.ds(c0, BM), :] += jnp.where(msk, wcol * eo, 0.0)


def _grouped_ffn(offs, sx, sw, W1, B1, W2, B2, W3, B3, SW1, SB1, SW2, SB2,
                 SW3, SB3):
    full = lambda shape: pl.BlockSpec(shape, lambda e, offs_ref: (0,) * len(shape))
    per_e3 = lambda shape: pl.BlockSpec(shape, lambda e, offs_ref: (e, 0, 0))
    grid_spec = pltpu.PrefetchScalarGridSpec(
        num_scalar_prefetch=1,
        grid=(E,),
        in_specs=[
            full((T, D)),
            full((T, 128)),
            per_e3((1, F, D)),
            per_e3((1, 1, F)),
            per_e3((1, D, F)),
            per_e3((1, 1, D)),
            per_e3((1, F, D)),
            per_e3((1, 1, F)),
            full((F, D)),
            full((1, F)),
            full((D, F)),
            full((1, D)),
            full((F, D)),
            full((1, F)),
        ],
        out_specs=full((T, D)),
    )
    return pl.pallas_call(
        _ffn_body,
        grid_spec=grid_spec,
        out_shape=jax.ShapeDtypeStruct((T, D), jnp.float32),
    )(offs, sx, sw, W1, B1.reshape(E, 1, F), W2, B2.reshape(E, 1, D),
      W3, B3.reshape(E, 1, F), SW1, SB1.reshape(1, F), SW2,
      SB2.reshape(1, D), SW3, SB3.reshape(1, F))


@jax.jit
def kernel(x, gate_w, W1, B1, W2, B2, W3, B3, SW1, SB1, SW2, SB2, SW3, SB3):
    sc_scatter, sc_gather = _sc_kernels()
    pos, offs2d, wrow = _gate_meta(x, gate_w)
    sx, sw = sc_scatter(x, wrow, pos)
    sy = _grouped_ffn(offs2d.reshape(2 * E), sx, sw, W1, B1, W2, B2, W3, B3,
                      SW1, SB1, SW2, SB2, SW3, SB3)
    return sc_gather(sy, pos)


# speedup vs baseline: 4.2581x; 4.2581x over previous
import functools

import jax
import jax.numpy as jnp
from jax import lax
from jax.experimental import pallas as pl
from jax.experimental.pallas import tpu as pltpu
from jax.experimental.pallas import tpu_sc as plsc

T = 2048
D = 768
F = 1024
E = 16
NW = 32
RPW = T // NW
BM = 128
NC = T // BM
CH = 512


def _silu(v):
    return v * (1.0 / (1.0 + jnp.exp(-v)))


def _gate_meta_body(x_ref, gw_ref, sw1_ref, sb1_ref, sw2_ref, sb2_ref,
                    sw3_ref, sb3_ref, pos_ref, offs_ref, wrow_ref, z_ref):
    x = x_ref[...]
    logits = jnp.dot(x, gw_ref[...].T, preferred_element_type=jnp.float32)
    m = jnp.max(logits, axis=1, keepdims=True)
    p = jnp.exp(logits - m)
    scores = p / jnp.sum(p, axis=1, keepdims=True)
    eidx = lax.broadcasted_iota(jnp.int32, (T, E), 1)
    first = jnp.min(jnp.where(logits == m, eidx, E), axis=1, keepdims=True)
    one_hot = (eidx == first).astype(jnp.float32)
    w = jnp.sum(scores * one_hot, axis=1, keepdims=True)
    wrow_ref[...] = jnp.broadcast_to(w, (T, 128))

    counts = jnp.sum(one_hot, axis=0, keepdims=True)
    iE = lax.broadcasted_iota(jnp.int32, (E, E), 0)
    jE = lax.broadcasted_iota(jnp.int32, (E, E), 1)
    Ue = (iE < jE).astype(jnp.float32)
    offs_row = jnp.dot(jnp.broadcast_to(counts, (8, E)), Ue,
                       preferred_element_type=jnp.float32)[0:1, :]
    Lc = (jE < iE).astype(jnp.float32)
    chunk_tot = jnp.sum(one_hot.reshape(NC, BM, E), axis=1)
    chunk_excl = jnp.dot(Lc, chunk_tot, preferred_element_type=jnp.float32)
    iB = lax.broadcasted_iota(jnp.int32, (BM, BM), 0)
    jB = lax.broadcasted_iota(jnp.int32, (BM, BM), 1)
    Lb = (jB < iB).astype(jnp.float32)
    for c in range(NC):
        oh_c = one_hot.reshape(NC, BM, E)[c]
        within = jnp.dot(Lb, oh_c, preferred_element_type=jnp.float32)
        rank = jnp.sum((within + chunk_excl[c:c + 1, :]) * oh_c, axis=1,
                       keepdims=True)
        base = jnp.sum(offs_row * oh_c, axis=1, keepdims=True)
        pos_c = (rank + base).astype(jnp.int32)
        rows_per_chunk = BM // RPW
        pos_ref[pl.ds(c * rows_per_chunk, rows_per_chunk), :] = (
            pos_c.reshape(rows_per_chunk, RPW))

    offs_pad = jnp.concatenate(
        [offs_row, jnp.full((1, E), float(T), jnp.float32)], axis=1)
    offs_ref[...] = offs_pad.astype(jnp.int32)

    for c in range(T // CH):
        xs = x_ref[pl.ds(c * CH, CH), :]
        h1 = jnp.dot(xs, sw1_ref[...].T, preferred_element_type=jnp.float32) + sb1_ref[...]
        h3 = jnp.dot(xs, sw3_ref[...].T, preferred_element_type=jnp.float32) + sb3_ref[...]
        z_ref[pl.ds(c * CH, CH), :] = (
            jnp.dot(_silu(h1) * h3, sw2_ref[...].T,
                    preferred_element_type=jnp.float32) + sb2_ref[...])


def _gate_meta(x, gate_w, SW1, SB1, SW2, SB2, SW3, SB3):
    return pl.pallas_call(
        _gate_meta_body,
        out_shape=(jax.ShapeDtypeStruct((NW, RPW), jnp.int32),
                   jax.ShapeDtypeStruct((1, 2 * E), jnp.int32),
                   jax.ShapeDtypeStruct((T, 128), jnp.float32),
                   jax.ShapeDtypeStruct((T, D), jnp.float32)),
    )(x, gate_w, SW1, SB1.reshape(1, F), SW2, SB2.reshape(1, D), SW3,
      SB3.reshape(1, F))


@functools.cache
def _sc_kernels():
    mesh = plsc.VectorSubcoreMesh(core_axis_name="c", subcore_axis_name="s")

    @functools.partial(
        pl.kernel,
        out_type=(jax.ShapeDtypeStruct((T, D), jnp.float32),
                  jax.ShapeDtypeStruct((T, D), jnp.float32),
                  jax.ShapeDtypeStruct((T, 128), jnp.float32)),
        mesh=mesh,
        scratch_types=[
            pltpu.VMEM((RPW,), jnp.int32),
            pltpu.VMEM((RPW, D), jnp.float32),
            pltpu.VMEM((RPW, D), jnp.float32),
            pltpu.VMEM((RPW, 128), jnp.float32),
            pltpu.SemaphoreType.DMA,
            pltpu.SemaphoreType.DMA,
            pltpu.SemaphoreType.DMA,
        ],
    )
    def _sc_scatter(x_hbm, z_hbm, w_hbm, pos_hbm, sx_hbm, sz_hbm, sw_hbm,
                    idx_v, x_v, z_v, w_v, s1, s2, s3):
        wid = lax.axis_index("s") * 2 + lax.axis_index("c")
        base = wid * RPW
        pltpu.sync_copy(pos_hbm.at[wid], idx_v)
        pltpu.sync_copy(x_hbm.at[pl.ds(base, RPW)], x_v)
        pltpu.sync_copy(z_hbm.at[pl.ds(base, RPW)], z_v)
        pltpu.sync_copy(w_hbm.at[pl.ds(base, RPW)], w_v)
        c1 = pltpu.async_copy(x_v, sx_hbm.at[idx_v], s1)
        c2 = pltpu.async_copy(z_v, sz_hbm.at[idx_v], s2)
        c3 = pltpu.async_copy(w_v, sw_hbm.at[idx_v], s3)
        c1.wait()
        c2.wait()
        c3.wait()

    @functools.partial(
        pl.kernel,
        out_type=jax.ShapeDtypeStruct((T, D), jnp.float32),
        mesh=mesh,
        scratch_types=[
            pltpu.VMEM((RPW,), jnp.int32),
            pltpu.VMEM((RPW, D), jnp.float32),
            pltpu.SemaphoreType.DMA,
        ],
    )
    def _sc_gather(sy_hbm, pos_hbm, out_hbm, idx_v, y_v, sem):
        wid = lax.axis_index("s") * 2 + lax.axis_index("c")
        base = wid * RPW
        pltpu.sync_copy(pos_hbm.at[wid], idx_v)
        pltpu.async_copy(sy_hbm.at[idx_v], y_v, sem).wait()
        pltpu.sync_copy(y_v, out_hbm.at[pl.ds(base, RPW)])

    return _sc_scatter, _sc_gather


def _ffn_body(offs_ref, sx_ref, sw_ref, sz_ref, w1_ref, b1_ref, w2_ref,
              b2_ref, w3_ref, b3_ref, out_ref):
    e = pl.program_id(0)

    @pl.when(e == 0)
    def _init():
        out_ref[...] = sz_ref[...]

    start = offs_ref[e]
    end = offs_ref[e + 1]
    for c in range(NC):
        c0 = c * BM

        @pl.when((end > c0) & (start < c0 + BM))
        def _chunk(c0=c0):
            xs = sx_ref[pl.ds(c0, BM), :]
            h1 = jnp.dot(xs, w1_ref[0].T, preferred_element_type=jnp.float32) + b1_ref[0]
            h3 = jnp.dot(xs, w3_ref[0].T, preferred_element_type=jnp.float32) + b3_ref[0]
            h = _silu(h1) * h3
            eo = jnp.dot(h, w2_ref[0].T, preferred_element_type=jnp.float32) + b2_ref[0]
            rows = c0 + lax.broadcasted_iota(jnp.int32, (BM, 1), 0)
            msk = (rows >= start) & (rows < end)
            wcol = sw_ref[pl.ds(c0, BM), 0:1]
            out_ref[pl.ds(c0, BM), :] += jnp.where(msk, wcol * eo, 0.0)


def _grouped_ffn(offs, sx, sw, sz, W1, B1, W2, B2, W3, B3):
    full = lambda shape: pl.BlockSpec(shape, lambda e, offs_ref: (0,) * len(shape))
    per_e3 = lambda shape: pl.BlockSpec(shape, lambda e, offs_ref: (e, 0, 0))
    grid_spec = pltpu.PrefetchScalarGridSpec(
        num_scalar_prefetch=1,
        grid=(E,),
        in_specs=[
            full((T, D)),
            full((T, 128)),
            full((T, D)),
            per_e3((1, F, D)),
            per_e3((1, 1, F)),
            per_e3((1, D, F)),
            per_e3((1, 1, D)),
            per_e3((1, F, D)),
            per_e3((1, 1, F)),
        ],
        out_specs=full((T, D)),
    )
    return pl.pallas_call(
        _ffn_body,
        grid_spec=grid_spec,
        out_shape=jax.ShapeDtypeStruct((T, D), jnp.float32),
    )(offs, sx, sw, sz, W1, B1.reshape(E, 1, F), W2, B2.reshape(E, 1, D),
      W3, B3.reshape(E, 1, F))


@jax.jit
def kernel(x, gate_w, W1, B1, W2, B2, W3, B3, SW1, SB1, SW2, SB2, SW3, SB3):
    sc_scatter, sc_gather = _sc_kernels()
    pos, offs2d, wrow, z = _gate_meta(x, gate_w, SW1, SB1, SW2, SB2, SW3, SB3)
    sx, sz, sw = sc_scatter(x, z, wrow, pos)
    sy = _grouped_ffn(offs2d.reshape(2 * E), sx, sw, sz, W1, B1, W2, B2, W3, B3)
    return sc_gather(sy, pos)
